# baseline (speedup 1.0000x reference)
import jax
import jax.numpy as jnp
from jax.experimental import pallas as pl
from jax.experimental.pallas import tpu as pltpu
from jax.experimental.pallas import tpu_sc as plsc

_B, _C, _N, _K = 4, 64, 2048, 16
_H, _O = 64, 128
_RB = 256
_NB = _N // _RB
_GB = (_B * _N) // _RB


def _leaky(x):
    return jnp.where(x >= 0, x, 0.2 * x)


def _batcher_pairs(n):
    pairs = []
    p = 1
    while p < n:
        k = p
        while k >= 1:
            for j in range(k % p, n - k, 2 * k):
                for i in range(0, min(k, n - j - k)):
                    if (i + j) // (2 * p) == (i + j + k) // (2 * p):
                        pairs.append((i + j, i + j + k))
            k //= 2
        p *= 2
    return pairs


_PAIRS16 = _batcher_pairs(16)


def _knn_kernel(xrow_ref, xall_ref, wa_ref, wz_ref, misc_ref,
                idx_ref, cf_ref, zf_ref):
    x = xrow_ref[0]
    xa = xall_ref[0]
    row_sq = jnp.sum(x * x, axis=1, keepdims=True)
    all_sq = jnp.sum(xa * xa, axis=1)[None, :]
    mm = jax.lax.dot_general(x, xa, (((1,), (1,)), ((), ())),
                             preferred_element_type=jnp.float32)
    dist = -2.0 * mm + row_sq + all_sq

    lane = jax.lax.broadcasted_iota(jnp.int32, (_RB, 128), 1)
    kiota = jax.lax.broadcasted_iota(jnp.int32, (_RB, _K), 1)
    vs = [dist[:, e * 128:(e + 1) * 128] for e in range(16)]
    iis = [lane + jnp.int32(128 * e) for e in range(16)]
    for (a, b) in _PAIRS16:
        cmp = vs[a] <= vs[b]
        va = jnp.where(cmp, vs[a], vs[b])
        vb = jnp.where(cmp, vs[b], vs[a])
        ia = jnp.where(cmp, iis[a], iis[b])
        ib = jnp.where(cmp, iis[b], iis[a])
        vs[a], vs[b], iis[a], iis[b] = va, vb, ia, ib
    vs.append(jnp.full((_RB, 128), jnp.inf, jnp.float32))
    iis.append(jnp.zeros((_RB, 128), jnp.int32))

    hv, hi = vs[0], iis[0]
    ptr = jnp.zeros((_RB, 128), jnp.int32)
    acc = jnp.zeros((_RB, _K), jnp.int32)
    for k in range(_K):
        m = jnp.min(hv, axis=1, keepdims=True)
        lsel = jnp.min(jnp.where(hv <= m, lane, 128), axis=1, keepdims=True)
        selmask = lane == lsel
        j = jnp.sum(jnp.where(selmask, hi, 0), axis=1, keepdims=True)
        acc = jnp.where(kiota == k, j, acc)
        ptr = ptr + selmask.astype(jnp.int32)
        nh, ni = vs[0], iis[0]
        for e in range(1, 17):
            pe = ptr == e
            nh = jnp.where(pe, vs[e], nh)
            ni = jnp.where(pe, iis[e], ni)
        hv = jnp.where(selmask, nh, hv)
        hi = jnp.where(selmask, ni, hi)
    idx_ref[0] = acc + pl.program_id(0) * _N
    b1 = misc_ref[0:1, :_H]
    cf_ref[0] = jnp.dot(x, wa_ref[...],
                        preferred_element_type=jnp.float32) + b1
    z = jnp.dot(x, wz_ref[...], preferred_element_type=jnp.float32)
    zf_ref[0] = jnp.concatenate([z, jnp.zeros((_RB, 128 - _H), z.dtype)],
                                axis=1)


_GW = 128


def _sc_gather(zf2, gidx):
    n_idx = gidx.shape[1]
    mesh = plsc.VectorSubcoreMesh(core_axis_name="core",
                                  subcore_axis_name="subcore")

    @pl.kernel(out_type=jax.ShapeDtypeStruct((n_idx, 128), zf2.dtype),
               mesh=mesh)
    def _gather_kernel(x_hbm, i_hbm, o_hbm):
        def body(i_vmem, o_vmem):
            pltpu.sync_copy(x_hbm.at[i_vmem.at[0]], o_vmem)

        pltpu.emit_pipeline(
            body,
            grid=(n_idx // _GW,),
            in_specs=[pl.BlockSpec((1, _GW), index_map=lambda i: (0, i))],
            out_specs=[pl.BlockSpec((_GW, 128), index_map=lambda i: (i, 0))],
            core_axis_name=("core", "subcore"),
            dimension_semantics=(pltpu.PARALLEL,),
        )(i_hbm, o_hbm)

    return _gather_kernel(zf2, gidx)


def _stats1_kernel(zg_ref, cf_ref, out_ref, acc_ref):
    g = pl.program_id(0)

    @pl.when(g == 0)
    def _():
        acc_ref[...] = jnp.zeros_like(acc_ref)

    x1 = cf_ref[...][:, None, :] - zg_ref[:, :, :_H]
    acc_ref[0:1, :_H] += jnp.sum(x1, axis=(0, 1))[None, :]
    acc_ref[1:2, :_H] += jnp.sum(x1 * x1, axis=(0, 1))[None, :]

    @pl.when(g == pl.num_programs(0) - 1)
    def _():
        out_ref[...] = acc_ref[...]


def _mid_kernel(zg_ref, cf_ref, w2_ref, misc_ref, out_ref, acc_ref):
    g = pl.program_id(0)

    @pl.when(g == 0)
    def _():
        acc_ref[...] = jnp.zeros_like(acc_ref)

    s1 = misc_ref[0:1, :_H]
    t1 = misc_ref[1:2, :_H]
    b2 = misc_ref[2:3, :_H]
    x1 = cf_ref[...][:, None, :] - zg_ref[:, :, :_H]
    y1 = _leaky(x1 * s1[None] + t1[None]).reshape(_RB * _K, _H)
    x2 = jnp.dot(y1.astype(jnp.bfloat16), w2_ref[...].astype(jnp.bfloat16),
                 preferred_element_type=jnp.float32) + b2
    acc_ref[0:1, :_H] += jnp.sum(x2, axis=0)[None, :]
    acc_ref[1:2, :_H] += jnp.sum(x2 * x2, axis=0)[None, :]

    @pl.when(g == pl.num_programs(0) - 1)
    def _():
        out_ref[...] = acc_ref[...]


def _final_kernel(zg_ref, cf_ref, w2_ref, w3_ref, misc_ref, out_ref):
    s1 = misc_ref[0:1, :_H]
    t1 = misc_ref[1:2, :_H]
    b2 = misc_ref[2:3, :_H]
    s2 = misc_ref[3:4, :_H]
    t2 = misc_ref[4:5, :_H]
    b3 = misc_ref[5:6, :_O]
    x1 = cf_ref[...][:, None, :] - zg_ref[:, :, :_H]
    y1 = _leaky(x1 * s1[None] + t1[None]).reshape(_RB * _K, _H)
    x2 = jnp.dot(y1.astype(jnp.bfloat16), w2_ref[...].astype(jnp.bfloat16),
                 preferred_element_type=jnp.float32) + b2
    y2 = _leaky(x2 * s2 + t2)
    x3 = jnp.dot(y2.astype(jnp.bfloat16), w3_ref[...].astype(jnp.bfloat16),
                 preferred_element_type=jnp.float32)
    x3 = x3.reshape(_RB, _K, _O)
    out_ref[...] = jnp.max(x3, axis=1) + b3


def _bn_coeffs(stats, gamma, beta, cnt):
    mean = stats[0, :_H] / cnt
    var = stats[1, :_H] / cnt - mean * mean
    s = gamma / jnp.sqrt(var + 1e-5)
    t = beta - mean * s
    return s, t


def _make_pipeline(lb):
    f32 = jnp.float32
    gridl = (lb * _N // _RB,)
    zg_spec = pl.BlockSpec((_RB, _K, 128), lambda g: (g, 0, 0))
    cf_spec = pl.BlockSpec((_RB, _H), lambda g: (g, 0))
    w_spec = pl.BlockSpec((_H, _H), lambda g: (0, 0))
    w3_spec = pl.BlockSpec((_H, _O), lambda g: (0, 0))
    misc_spec = pl.BlockSpec((8, 128), lambda g: (0, 0))
    stat_spec = pl.BlockSpec((8, 128), lambda g: (0, 0))
    stat_shape = jax.ShapeDtypeStruct((8, 128), f32)
    scratch = [pltpu.VMEM((8, 128), f32)]

    def fn(xyz_l, Wa, Wz, misc1, W2T, W3T, g1, be1, b2, g2, be2, b3):
        xyzT = jnp.transpose(xyz_l, (0, 2, 1))
        idx, cf, zf = pl.pallas_call(
            _knn_kernel,
            grid=(lb, _NB),
            in_specs=[
                pl.BlockSpec((1, _RB, _C), lambda b, i: (b, i, 0)),
                pl.BlockSpec((1, _N, _C), lambda b, i: (b, 0, 0)),
                pl.BlockSpec((_C, _H), lambda b, i: (0, 0)),
                pl.BlockSpec((_C, _H), lambda b, i: (0, 0)),
                pl.BlockSpec((8, 128), lambda b, i: (0, 0)),
            ],
            out_specs=[
                pl.BlockSpec((1, _RB, _K), lambda b, i: (b, i, 0)),
                pl.BlockSpec((1, _RB, _H), lambda b, i: (b, i, 0)),
                pl.BlockSpec((1, _RB, 128), lambda b, i: (b, i, 0)),
            ],
            out_shape=[
                jax.ShapeDtypeStruct((lb, _N, _K), jnp.int32),
                jax.ShapeDtypeStruct((lb, _N, _H), f32),
                jax.ShapeDtypeStruct((lb, _N, 128), f32),
            ],
        )(xyzT, xyzT, Wa, Wz, misc1)

        zf2 = zf.reshape(lb * _N, 128)
        gidx = idx.reshape(1, lb * _N * _K)
        zg_rows = _sc_gather(zf2, gidx)
        zgf = zg_rows.reshape(lb * _N, _K, 128)
        cff = cf.reshape(lb * _N, _H)

        st1 = pl.pallas_call(
            _stats1_kernel, grid=gridl,
            in_specs=[zg_spec, cf_spec], out_specs=stat_spec,
            out_shape=stat_shape, scratch_shapes=scratch,
        )(zgf, cff)


        cnt = float(_B * _N * _K)
        s1, t1 = _bn_coeffs(st1, g1, be1, cnt)
        misc2 = (jnp.zeros((8, 128), f32)
                 .at[0, :_H].set(s1).at[1, :_H].set(t1).at[2, :_H].set(b2))

        st2 = pl.pallas_call(
            _mid_kernel, grid=gridl,
            in_specs=[zg_spec, cf_spec, w_spec, misc_spec],
            out_specs=stat_spec,
            out_shape=stat_shape, scratch_shapes=scratch,
        )(zgf, cff, W2T, misc2)


        s2, t2 = _bn_coeffs(st2, g2, be2, cnt)
        misc3 = (misc2.at[3, :_H].set(s2).at[4, :_H].set(t2)
                 .at[5, :_O].set(b3))

        outf = pl.pallas_call(
            _final_kernel, grid=gridl,
            in_specs=[zg_spec, cf_spec, w_spec, w3_spec, misc_spec],
            out_specs=pl.BlockSpec((_RB, _O), lambda g: (g, 0)),
            out_shape=jax.ShapeDtypeStruct((lb * _N, _O), f32),
        )(zgf, cff, W2T, W3T, misc3)

        return outf.reshape(lb, _N, _O).transpose(0, 2, 1)

    return fn


def kernel(inputs, W1, b1, g1, be1, W2, b2, g2, be2, W3, b3):
    f32 = jnp.float32
    W1e = W1[:, :_C]
    Wa = jnp.transpose(W1e + W1[:, _C:])
    Wz = jnp.transpose(W1e)
    misc1 = jnp.zeros((8, 128), f32).at[0, :_H].set(b1)
    W2T = jnp.transpose(W2)
    W3T = jnp.transpose(W3)

    return _make_pipeline(_B)(inputs, Wa, Wz, misc1, W2T, W3T,
                              g1, be1, b2, g2, be2, b3)


# speedup vs baseline: 1.0578x; 1.0578x over previous
import jax
import jax.numpy as jnp
from jax.experimental import pallas as pl
from jax.experimental.pallas import tpu as pltpu
from jax.experimental.pallas import tpu_sc as plsc

_B, _C, _N, _K = 4, 64, 2048, 16
_H, _O = 64, 128
_RB = 256
_NB = _N // _RB
_GB = (_B * _N) // _RB


def _leaky(x):
    return jnp.where(x >= 0, x, 0.2 * x)


def _batcher_pairs(n):
    pairs = []
    p = 1
    while p < n:
        k = p
        while k >= 1:
            for j in range(k % p, n - k, 2 * k):
                for i in range(0, min(k, n - j - k)):
                    if (i + j) // (2 * p) == (i + j + k) // (2 * p):
                        pairs.append((i + j, i + j + k))
            k //= 2
        p *= 2
    return pairs


_PAIRS16 = _batcher_pairs(16)


def _knn_kernel(xrow_ref, xall_ref, wa_ref, wz_ref, misc_ref,
                idx_ref, cf_ref, zf_ref):
    x = xrow_ref[0]
    xa = xall_ref[0]
    row_sq = jnp.sum(x * x, axis=1, keepdims=True)
    all_sq = jnp.sum(xa * xa, axis=1)[None, :]
    mm = jax.lax.dot_general(x, xa, (((1,), (1,)), ((), ())),
                             preferred_element_type=jnp.float32)
    dist = -2.0 * mm + row_sq + all_sq

    lane = jax.lax.broadcasted_iota(jnp.int32, (_RB, 128), 1)
    kiota = jax.lax.broadcasted_iota(jnp.int32, (_RB, _K), 1)
    vs = [dist[:, e * 128:(e + 1) * 128] for e in range(16)]
    iis = [lane + jnp.int32(128 * e) for e in range(16)]
    for (a, b) in _PAIRS16:
        cmp = vs[a] <= vs[b]
        va = jnp.where(cmp, vs[a], vs[b])
        vb = jnp.where(cmp, vs[b], vs[a])
        ia = jnp.where(cmp, iis[a], iis[b])
        ib = jnp.where(cmp, iis[b], iis[a])
        vs[a], vs[b], iis[a], iis[b] = va, vb, ia, ib
    vs.append(jnp.full((_RB, 128), jnp.inf, jnp.float32))
    iis.append(jnp.zeros((_RB, 128), jnp.int32))

    hv, hi = vs[0], iis[0]
    ptr = jnp.zeros((_RB, 128), jnp.int32)
    acc = jnp.zeros((_RB, _K), jnp.int32)
    for k in range(_K):
        m = jnp.min(hv, axis=1, keepdims=True)
        lsel = jnp.min(jnp.where(hv <= m, lane, 128), axis=1, keepdims=True)
        selmask = lane == lsel
        j = jnp.sum(jnp.where(selmask, hi, 0), axis=1, keepdims=True)
        acc = jnp.where(kiota == k, j, acc)
        ptr = ptr + selmask.astype(jnp.int32)
        nh, ni = vs[0], iis[0]
        for e in range(1, 17):
            pe = ptr == e
            nh = jnp.where(pe, vs[e], nh)
            ni = jnp.where(pe, iis[e], ni)
        hv = jnp.where(selmask, nh, hv)
        hi = jnp.where(selmask, ni, hi)
    idx_ref[0] = acc + pl.program_id(0) * _N
    b1 = misc_ref[0:1, :_H]
    cf_ref[0] = jnp.dot(x, wa_ref[...],
                        preferred_element_type=jnp.float32) + b1
    z = jnp.dot(x, wz_ref[...], preferred_element_type=jnp.float32)
    zf_ref[0] = jnp.concatenate([z, jnp.zeros((_RB, 128 - _H), z.dtype)],
                                axis=1)


_GW = 128


def _sc_gather(zf2, gidx):
    n_idx = gidx.shape[1]
    mesh = plsc.VectorSubcoreMesh(core_axis_name="core",
                                  subcore_axis_name="subcore")

    @pl.kernel(out_type=jax.ShapeDtypeStruct((n_idx, 128), zf2.dtype),
               mesh=mesh)
    def _gather_kernel(x_hbm, i_hbm, o_hbm):
        def body(i_vmem, o_vmem):
            pltpu.sync_copy(x_hbm.at[i_vmem.at[0]], o_vmem)

        pltpu.emit_pipeline(
            body,
            grid=(n_idx // _GW,),
            in_specs=[pl.BlockSpec((1, _GW), index_map=lambda i: (0, i))],
            out_specs=[pl.BlockSpec((_GW, 128), index_map=lambda i: (i, 0))],
            core_axis_name=("core", "subcore"),
            dimension_semantics=(pltpu.PARALLEL,),
        )(i_hbm, o_hbm)

    return _gather_kernel(zf2, gidx)


def _conv_kernel(zg_ref, cf_ref, w2_ref, w3_ref, misc_ref, out_ref,
                 x1_ref, acc1_ref, acc2_ref, coef_ref):
    p = pl.program_id(0)
    g = pl.program_id(1)
    cnt = jnp.float32(_B * _N * _K)
    bf16 = jnp.bfloat16

    @pl.when((p == 0) & (g == 0))
    def _():
        acc1_ref[...] = jnp.zeros_like(acc1_ref)
        acc2_ref[...] = jnp.zeros_like(acc2_ref)

    @pl.when(p == 0)
    def _():
        x1 = cf_ref[...][:, None, :] - zg_ref[:, :, :_H]
        x1_ref[pl.ds(g * _RB, _RB)] = x1.astype(jnp.bfloat16)
        acc1_ref[0:1, :_H] += jnp.sum(x1, axis=(0, 1))[None, :]
        acc1_ref[1:2, :_H] += jnp.sum(x1 * x1, axis=(0, 1))[None, :]

    @pl.when((p == 1) & (g == 0))
    def _():
        mean = acc1_ref[0:1, :_H] / cnt
        var = acc1_ref[1:2, :_H] / cnt - mean * mean
        s = misc_ref[1:2, :_H] / jnp.sqrt(var + 1e-5)
        coef_ref[0:1, :_H] = s
        coef_ref[1:2, :_H] = misc_ref[2:3, :_H] - mean * s

    def _x2(x1):
        x1 = x1.astype(jnp.float32)
        y1 = _leaky(x1 * coef_ref[0:1, :_H][None] + coef_ref[1:2, :_H][None])
        return jnp.dot(y1.reshape(_RB * _K, _H).astype(bf16),
                       w2_ref[...].astype(bf16),
                       preferred_element_type=jnp.float32) + misc_ref[0:1, :_H]

    @pl.when(p == 1)
    def _():
        x2 = _x2(x1_ref[pl.ds(g * _RB, _RB)])
        acc2_ref[0:1, :_H] += jnp.sum(x2, axis=0)[None, :]
        acc2_ref[1:2, :_H] += jnp.sum(x2 * x2, axis=0)[None, :]

    @pl.when((p == 2) & (g == 0))
    def _():
        mean = acc2_ref[0:1, :_H] / cnt
        var = acc2_ref[1:2, :_H] / cnt - mean * mean
        s = misc_ref[3:4, :_H] / jnp.sqrt(var + 1e-5)
        coef_ref[2:3, :_H] = s
        coef_ref[3:4, :_H] = misc_ref[4:5, :_H] - mean * s

    @pl.when(p == 2)
    def _():
        x2 = _x2(x1_ref[pl.ds(g * _RB, _RB)])
        y2 = _leaky(x2 * coef_ref[2:3, :_H] + coef_ref[3:4, :_H])
        x3 = jnp.dot(y2.astype(bf16), w3_ref[...].astype(bf16),
                     preferred_element_type=jnp.float32)
        x3 = x3.reshape(_RB, _K, _O)
        out_ref[...] = jnp.max(x3, axis=1) + misc_ref[5:6, :_O]


def _bn_coeffs(stats, gamma, beta, cnt):
    mean = stats[0, :_H] / cnt
    var = stats[1, :_H] / cnt - mean * mean
    s = gamma / jnp.sqrt(var + 1e-5)
    t = beta - mean * s
    return s, t


def _make_pipeline(lb):
    f32 = jnp.float32
    nblk = lb * _N // _RB
    zg_spec = pl.BlockSpec((_RB, _K, 128),
                           lambda p, g: (jnp.where(p == 0, g, 0), 0, 0))
    cf_spec = pl.BlockSpec((_RB, _H),
                           lambda p, g: (jnp.where(p == 0, g, 0), 0))
    w_spec = pl.BlockSpec((_H, _H), lambda p, g: (0, 0))
    w3_spec = pl.BlockSpec((_H, _O), lambda p, g: (0, 0))
    misc_spec = pl.BlockSpec((8, 128), lambda p, g: (0, 0))

    def fn(xyz_l, Wa, Wz, misc1, W2T, W3T, g1, be1, b2, g2, be2, b3):
        xyzT = jnp.transpose(xyz_l, (0, 2, 1))
        idx, cf, zf = pl.pallas_call(
            _knn_kernel,
            grid=(lb, _NB),
            in_specs=[
                pl.BlockSpec((1, _RB, _C), lambda b, i: (b, i, 0)),
                pl.BlockSpec((1, _N, _C), lambda b, i: (b, 0, 0)),
                pl.BlockSpec((_C, _H), lambda b, i: (0, 0)),
                pl.BlockSpec((_C, _H), lambda b, i: (0, 0)),
                pl.BlockSpec((8, 128), lambda b, i: (0, 0)),
            ],
            out_specs=[
                pl.BlockSpec((1, _RB, _K), lambda b, i: (b, i, 0)),
                pl.BlockSpec((1, _RB, _H), lambda b, i: (b, i, 0)),
                pl.BlockSpec((1, _RB, 128), lambda b, i: (b, i, 0)),
            ],
            out_shape=[
                jax.ShapeDtypeStruct((lb, _N, _K), jnp.int32),
                jax.ShapeDtypeStruct((lb, _N, _H), f32),
                jax.ShapeDtypeStruct((lb, _N, 128), f32),
            ],
        )(xyzT, xyzT, Wa, Wz, misc1)

        zf2 = zf.reshape(lb * _N, 128)
        gidx = idx.reshape(1, lb * _N * _K)
        zg_rows = _sc_gather(zf2, gidx)
        zgf = zg_rows.reshape(lb * _N, _K, 128)
        cff = cf.reshape(lb * _N, _H)

        miscc = (jnp.zeros((8, 128), f32)
                 .at[0, :_H].set(b2).at[1, :_H].set(g1).at[2, :_H].set(be1)
                 .at[3, :_H].set(g2).at[4, :_H].set(be2).at[5, :_O].set(b3))

        outf = pl.pallas_call(
            _conv_kernel, grid=(3, nblk),
            in_specs=[zg_spec, cf_spec, w_spec, w3_spec, misc_spec],
            out_specs=pl.BlockSpec(
                (_RB, _O), lambda p, g: (jnp.where(p == 2, g, 0), 0)),
            out_shape=jax.ShapeDtypeStruct((lb * _N, _O), f32),
            scratch_shapes=[pltpu.VMEM((lb * _N, _K, _H), jnp.bfloat16),
                            pltpu.VMEM((8, 128), f32),
                            pltpu.VMEM((8, 128), f32),
                            pltpu.VMEM((8, 128), f32)],
        )(zgf, cff, W2T, W3T, miscc)

        return outf.reshape(lb, _N, _O).transpose(0, 2, 1)

    return fn


def kernel(inputs, W1, b1, g1, be1, W2, b2, g2, be2, W3, b3):
    f32 = jnp.float32
    W1e = W1[:, :_C]
    Wa = jnp.transpose(W1e + W1[:, _C:])
    Wz = jnp.transpose(W1e)
    misc1 = jnp.zeros((8, 128), f32).at[0, :_H].set(b1)
    W2T = jnp.transpose(W2)
    W3T = jnp.transpose(W3)

    return _make_pipeline(_B)(inputs, Wa, Wz, misc1, W2T, W3T,
                              g1, be1, b2, g2, be2, b3)
